# Initial kernel scaffold; baseline (speedup 1.0000x reference)
import jax
import jax.numpy as jnp
from jax.experimental import pallas as pl


def kernel(x, edge_index, st1_tc1_w, st1_tc1_b, st1_cheb_w, st1_cheb_b, st1_tc2_w, st1_tc2_b, st1_bn_g, st1_bn_b, st2_tc1_w, st2_tc1_b, st2_cheb_w, st2_cheb_b, st2_tc2_w, st2_tc2_b, st2_bn_g, st2_bn_b, fc_w, fc_b):
    raise NotImplementedError("write your pallas kernel here")


# speedup vs baseline: 9.4745x; 9.4745x over previous
import functools

import jax
import jax.numpy as jnp
from jax import lax
from jax.experimental import pallas as pl
from jax.experimental.pallas import tpu as pltpu
from jax.experimental.pallas import tpu_sc as plsc

N = 10000
B = 2
T = 12
E = 160000
HID = 16
OUT = 64
NCLS = 10

NT = 1000
NB = N // NT
NP = 10112
NW = 32
STRIPE = NP // 16
EP = 163840
EPW = EP // NW
NCH = EPW // 128


def _make_scatter(S):
    mesh = plsc.VectorSubcoreMesh(core_axis_name="c", subcore_axis_name="s")

    @functools.partial(
        pl.kernel,
        mesh=mesh,
        compiler_params=pltpu.CompilerParams(use_tc_tiling_on_sc=False),
        out_type=jax.ShapeDtypeStruct((2, S, NP, 16), jnp.float32),
        scratch_types=[
            pltpu.VMEM((NCH, 128), jnp.int32),
            pltpu.VMEM((NCH, 128), jnp.int32),
            pltpu.VMEM((NCH, 128, 16), jnp.float32),
            pltpu.VMEM((STRIPE, 16), jnp.float32),
            pltpu.VMEM_SHARED((NP, 16), jnp.float32),
            pltpu.SemaphoreType.DMA,
        ],
    )
    def scat(yf, srcf, dstp, zeros_hbm, parts, src_v, dst_v, rows_v, zeros_v,
             acc, sem):
        cid = lax.axis_index("c")
        sid = lax.axis_index("s")
        wid = cid * 16 + sid
        pltpu.sync_copy(dstp.at[wid], dst_v)
        pltpu.sync_copy(zeros_hbm, zeros_v)
        for s in range(S):
            pltpu.sync_copy(zeros_v, acc.at[pl.ds(sid * STRIPE, STRIPE)])
            pltpu.sync_copy(srcf.at[s, wid], src_v)
            plsc.subcore_barrier()

            def body(j, carry):
                pltpu.async_copy(yf.at[src_v.at[j]], rows_v.at[j], sem).wait()
                pltpu.sync_copy(rows_v.at[j], acc.at[dst_v.at[j]], add=True)
                return carry

            lax.fori_loop(0, NCH, body, 0)
            plsc.subcore_barrier()
            pltpu.sync_copy(acc.at[pl.ds(sid * STRIPE, STRIPE)],
                            parts.at[cid, s, pl.ds(sid * STRIPE, STRIPE)])
            plsc.subcore_barrier()

    return scat


def _scatter(y3, srcf, dstp, zeros, S):
    yf = y3.reshape(S * N, 16)
    return _make_scatter(S)(yf, srcf, dstp, zeros)


def _tconv(X, w, b, dis2):
    Bb, Tc, _, C = X.shape
    Cout = w.shape[1]
    To = Tc - 2
    wk = jnp.transpose(w[:, :, :, 0, :], (0, 3, 2, 1))
    want_y = dis2 is not None

    def body(x_ref, w_ref, b_ref, d_ref, t_ref, *maybe_y):
        t = pl.program_id(2)
        xs = [jnp.reshape(x_ref[0, pl.ds(t + k, 1)], (NT, C)) for k in range(3)]

        def conv(g):
            acc = jnp.zeros((NT, Cout), jnp.float32)
            for k in range(3):
                acc = acc + lax.dot_general(
                    xs[k], w_ref[g, k], (((1,), (0,)), ((), ())),
                    preferred_element_type=jnp.float32)
            return acc + b_ref[g][None, :]

        P = conv(0)
        Q = jax.nn.sigmoid(conv(1))
        R = conv(2)
        H = jnp.maximum(P * Q + R, 0.0)
        t_ref[0, 0] = H
        if maybe_y:
            maybe_y[0][0, 0] = d_ref[:] * H

    out_shapes = [jax.ShapeDtypeStruct((Bb, To, N, Cout), jnp.float32)]
    if want_y:
        out_shapes.append(jax.ShapeDtypeStruct((Bb, To, N, Cout), jnp.float32))
    d_in = dis2 if want_y else jnp.zeros((N, 1), jnp.float32)
    res = pl.pallas_call(
        body,
        grid=(Bb, NB, To),
        in_specs=[
            pl.BlockSpec((1, Tc, NT, C), lambda bb, nn, tt: (bb, 0, nn, 0)),
            pl.BlockSpec((3, 3, C, Cout), lambda bb, nn, tt: (0, 0, 0, 0)),
            pl.BlockSpec((3, Cout), lambda bb, nn, tt: (0, 0)),
            pl.BlockSpec((NT, 1), lambda bb, nn, tt: (nn, 0)),
        ],
        out_specs=[pl.BlockSpec((1, 1, NT, Cout), lambda bb, nn, tt: (bb, tt, nn, 0))
                   for _ in out_shapes],
        out_shape=out_shapes,
    )(X, wk, b, d_in)
    return res if want_y else (res[0], None)


def _combine_a(parts, dis2, S):
    def body(p_ref, d_ref, tx1_ref, y1_ref):
        a = p_ref[0, 0] + p_ref[1, 0]
        d = d_ref[:]
        tx1 = -d * a
        tx1_ref[0] = tx1
        y1_ref[0] = d * tx1

    return pl.pallas_call(
        body,
        grid=(S, NB),
        in_specs=[
            pl.BlockSpec((2, 1, NT, 16), lambda s, nn: (0, s, nn, 0)),
            pl.BlockSpec((NT, 1), lambda s, nn: (nn, 0)),
        ],
        out_specs=[pl.BlockSpec((1, NT, 16), lambda s, nn: (s, nn, 0))] * 2,
        out_shape=[jax.ShapeDtypeStruct((S, N, 16), jnp.float32)] * 2,
    )(parts, dis2)


def _combine_b(parts, tx0, tx1, dis2, W, bvec, S):
    def body(p_ref, x0_ref, x1_ref, d_ref, w_ref, b_ref, o_ref):
        a2 = p_ref[0, 0] + p_ref[1, 0]
        x0 = x0_ref[0]
        x1 = x1_ref[0]
        x2 = -2.0 * d_ref[:] * a2 - x0
        dn = (((1,), (0,)), ((), ()))
        o = (lax.dot_general(x0, w_ref[0], dn, preferred_element_type=jnp.float32)
             + lax.dot_general(x1, w_ref[1], dn, preferred_element_type=jnp.float32)
             + lax.dot_general(x2, w_ref[2], dn, preferred_element_type=jnp.float32)
             + b_ref[:])
        o_ref[0] = jnp.maximum(o, 0.0)

    return pl.pallas_call(
        body,
        grid=(S, NB),
        in_specs=[
            pl.BlockSpec((2, 1, NT, 16), lambda s, nn: (0, s, nn, 0)),
            pl.BlockSpec((1, NT, 16), lambda s, nn: (s, nn, 0)),
            pl.BlockSpec((1, NT, 16), lambda s, nn: (s, nn, 0)),
            pl.BlockSpec((NT, 1), lambda s, nn: (nn, 0)),
            pl.BlockSpec((3, 16, 16), lambda s, nn: (0, 0, 0)),
            pl.BlockSpec((1, 16), lambda s, nn: (0, 0)),
        ],
        out_specs=pl.BlockSpec((1, NT, 16), lambda s, nn: (s, nn, 0)),
        out_shape=jax.ShapeDtypeStruct((S, N, 16), jnp.float32),
    )(parts, tx0, tx1, dis2, W, bvec)


def _bn_elu(t4, g, b):
    Bb, To, _, C = t4.shape
    x3 = t4.reshape(Bb * To, N, C)
    M = Bb * To
    inv_cnt = 1.0 / (M * C)

    def body(x_ref, g_ref, b_ref, o_ref):
        x = x_ref[...]
        s = jnp.sum(jnp.sum(x, axis=2, keepdims=True), axis=0, keepdims=True)
        s2 = jnp.sum(jnp.sum(x * x, axis=2, keepdims=True), axis=0, keepdims=True)
        m = s * inv_cnt
        v = s2 * inv_cnt - m * m
        xn = (x - m) * lax.rsqrt(v + 1e-5)
        xn = xn * g_ref[:][None] + b_ref[:][None]
        o_ref[...] = jnp.where(xn > 0.0, xn, jnp.exp(xn) - 1.0)

    out = pl.pallas_call(
        body,
        grid=(NB,),
        in_specs=[
            pl.BlockSpec((M, NT, C), lambda nn: (0, nn, 0)),
            pl.BlockSpec((NT, 1), lambda nn: (nn, 0)),
            pl.BlockSpec((NT, 1), lambda nn: (nn, 0)),
        ],
        out_specs=pl.BlockSpec((M, NT, C), lambda nn: (0, nn, 0)),
        out_shape=jax.ShapeDtypeStruct((M, N, C), jnp.float32),
    )(x3, g.reshape(N, 1), b.reshape(N, 1))
    return out.reshape(Bb, To, N, C)


def _fc(h2, wpad):
    K = h2.shape[1]
    KC = 128000
    G = K // KC

    def body(x_ref, w_ref, o_ref):
        @pl.when(pl.program_id(0) == 0)
        def _():
            o_ref[...] = jnp.zeros((8, 16), jnp.float32)

        y = lax.dot_general(x_ref[...], w_ref[...], (((1,), (1,)), ((), ())),
                            preferred_element_type=jnp.float32)
        o_ref[...] = o_ref[...] + jnp.concatenate(
            [y, jnp.zeros((6, 16), jnp.float32)], axis=0)

    return pl.pallas_call(
        body,
        grid=(G,),
        in_specs=[
            pl.BlockSpec((B, KC), lambda k: (0, k)),
            pl.BlockSpec((16, KC), lambda k: (0, k)),
        ],
        out_specs=pl.BlockSpec((8, 16), lambda k: (0, 0)),
        out_shape=jax.ShapeDtypeStruct((8, 16), jnp.float32),
    )(h2, wpad)


def _cheb_block(x4, W, bvec, dis2, y4, srcf, dstp, zeros):
    Bb, Tc, _, F = x4.shape
    S = Bb * Tc
    tx0 = x4.reshape(S, N, F)
    y0 = y4.reshape(S, N, F)
    parts1 = _scatter(y0, srcf[:S], dstp, zeros, S)
    tx1, y1 = _combine_a(parts1, dis2, S)
    parts2 = _scatter(y1, srcf[:S], dstp, zeros, S)
    out = _combine_b(parts2, tx0, tx1, dis2, W, bvec, S)
    return out.reshape(Bb, Tc, N, F)


def kernel(x, edge_index, st1_tc1_w, st1_tc1_b, st1_cheb_w, st1_cheb_b,
           st1_tc2_w, st1_tc2_b, st1_bn_g, st1_bn_b, st2_tc1_w, st2_tc1_b,
           st2_cheb_w, st2_cheb_b, st2_tc2_w, st2_tc2_b, st2_bn_g, st2_bn_b,
           fc_w, fc_b):
    ei = edge_index.astype(jnp.int32)
    src, dst = ei[0], ei[1]
    ew = src != dst
    deg = jnp.zeros((N,), jnp.float32).at[src].add(ew.astype(jnp.float32))
    dis = jnp.where(deg > 0, lax.rsqrt(jnp.where(deg > 0, deg, 1.0)), 0.0)
    dis2 = dis.reshape(N, 1)

    pad = EP - E
    src_p = jnp.concatenate([src, jnp.zeros((pad,), jnp.int32)])
    dst_p = jnp.concatenate([jnp.where(ew, dst, N), jnp.full((pad,), N, jnp.int32)])
    dstp = dst_p.reshape(NW, NCH, 128)
    S1 = B * (T - 2)
    srcf = (src_p[None, :] + (jnp.arange(S1, dtype=jnp.int32) * N)[:, None])
    srcf = srcf.reshape(S1, NW, NCH, 128)
    zeros = jnp.zeros((STRIPE, 16), jnp.float32)

    t1, y1 = _tconv(x, st1_tc1_w, st1_tc1_b, dis2)
    c1 = _cheb_block(t1, st1_cheb_w, st1_cheb_b.reshape(1, HID), dis2, y1,
                     srcf, dstp, zeros)
    t2, _ = _tconv(c1, st1_tc2_w, st1_tc2_b, None)
    h = _bn_elu(t2, st1_bn_g, st1_bn_b)

    t3, y3 = _tconv(h, st2_tc1_w, st2_tc1_b, dis2)
    c2 = _cheb_block(t3, st2_cheb_w, st2_cheb_b.reshape(1, HID), dis2, y3,
                     srcf, dstp, zeros)
    t4, _ = _tconv(c2, st2_tc2_w, st2_tc2_b, None)
    h = _bn_elu(t4, st2_bn_g, st2_bn_b)

    h2 = h.reshape(B, -1)
    wpad = jnp.concatenate(
        [fc_w, jnp.zeros((16 - NCLS, fc_w.shape[1]), jnp.float32)], axis=0)
    logits = _fc(h2, wpad)
    return logits[:B, :NCLS] + fc_b[None, :]


# speedup vs baseline: 10.9048x; 1.1510x over previous
import functools

import jax
import jax.numpy as jnp
from jax import lax
from jax.experimental import pallas as pl
from jax.experimental.pallas import tpu as pltpu
from jax.experimental.pallas import tpu_sc as plsc

N = 10000
B = 2
T = 12
E = 160000
HID = 16
OUT = 64
NCLS = 10

NT = 1000
NB = N // NT
NP = 10112
NW = 32
STRIPE = NP // 16
EP = 163840
EPW = EP // NW
CW = 512
NCH = EPW // CW


def _make_scatter(S):
    mesh = plsc.VectorSubcoreMesh(core_axis_name="c", subcore_axis_name="s")

    @functools.partial(
        pl.kernel,
        mesh=mesh,
        compiler_params=pltpu.CompilerParams(use_tc_tiling_on_sc=False),
        out_type=jax.ShapeDtypeStruct((2, S, NP, 16), jnp.float32),
        scratch_types=[
            pltpu.VMEM((NCH, CW), jnp.int32),
            pltpu.VMEM((NCH, CW), jnp.int32),
            pltpu.VMEM((NCH, CW, 16), jnp.float32),
            pltpu.VMEM((STRIPE, 16), jnp.float32),
            pltpu.VMEM_SHARED((NP, 16), jnp.float32),
            pltpu.SemaphoreType.DMA,
        ],
    )
    def scat(yf, srcf, dstp, zeros_hbm, parts, src_v, dst_v, rows_v, zeros_v,
             acc, sem):
        cid = lax.axis_index("c")
        sid = lax.axis_index("s")
        wid = cid * 16 + sid
        pltpu.sync_copy(dstp.at[wid], dst_v)
        pltpu.sync_copy(zeros_hbm, zeros_v)
        for s in range(S):
            pltpu.sync_copy(zeros_v, acc.at[pl.ds(sid * STRIPE, STRIPE)])
            pltpu.sync_copy(srcf.at[s, wid], src_v)
            plsc.subcore_barrier()

            def body(j, carry):
                pltpu.async_copy(yf.at[src_v.at[j]], rows_v.at[j], sem).wait()
                pltpu.sync_copy(rows_v.at[j], acc.at[dst_v.at[j]], add=True)
                return carry

            lax.fori_loop(0, NCH, body, 0)
            plsc.subcore_barrier()
            pltpu.sync_copy(acc.at[pl.ds(sid * STRIPE, STRIPE)],
                            parts.at[cid, s, pl.ds(sid * STRIPE, STRIPE)])
            plsc.subcore_barrier()

    return scat


def _scatter(y3, srcf, dstp, zeros, S):
    yf = y3.reshape(S * N, 16)
    return _make_scatter(S)(yf, srcf, dstp, zeros)


def _tconv(X, w, b, dis2):
    Bb, Tc, _, C = X.shape
    Cout = w.shape[1]
    To = Tc - 2
    wk = jnp.transpose(w[:, :, :, 0, :], (0, 3, 2, 1))
    want_y = dis2 is not None

    def body(x_ref, w_ref, b_ref, d_ref, t_ref, *maybe_y):
        t = pl.program_id(2)
        xs = [jnp.reshape(x_ref[0, pl.ds(t + k, 1)], (NT, C)) for k in range(3)]

        def conv(g):
            acc = jnp.zeros((NT, Cout), jnp.float32)
            for k in range(3):
                acc = acc + lax.dot_general(
                    xs[k], w_ref[g, k], (((1,), (0,)), ((), ())),
                    preferred_element_type=jnp.float32)
            return acc + b_ref[g][None, :]

        P = conv(0)
        Q = jax.nn.sigmoid(conv(1))
        R = conv(2)
        H = jnp.maximum(P * Q + R, 0.0)
        t_ref[0, 0] = H
        if maybe_y:
            maybe_y[0][0, 0] = d_ref[:] * H

    out_shapes = [jax.ShapeDtypeStruct((Bb, To, N, Cout), jnp.float32)]
    if want_y:
        out_shapes.append(jax.ShapeDtypeStruct((Bb, To, N, Cout), jnp.float32))
    d_in = dis2 if want_y else jnp.zeros((N, 1), jnp.float32)
    res = pl.pallas_call(
        body,
        grid=(Bb, NB, To),
        in_specs=[
            pl.BlockSpec((1, Tc, NT, C), lambda bb, nn, tt: (bb, 0, nn, 0)),
            pl.BlockSpec((3, 3, C, Cout), lambda bb, nn, tt: (0, 0, 0, 0)),
            pl.BlockSpec((3, Cout), lambda bb, nn, tt: (0, 0)),
            pl.BlockSpec((NT, 1), lambda bb, nn, tt: (nn, 0)),
        ],
        out_specs=[pl.BlockSpec((1, 1, NT, Cout), lambda bb, nn, tt: (bb, tt, nn, 0))
                   for _ in out_shapes],
        out_shape=out_shapes,
    )(X, wk, b, d_in)
    return res if want_y else (res[0], None)


def _combine_a(parts, dis2, S):
    def body(p_ref, d_ref, tx1_ref, y1_ref):
        a = p_ref[0, 0] + p_ref[1, 0]
        d = d_ref[:]
        tx1 = -d * a
        tx1_ref[0] = tx1
        y1_ref[0] = d * tx1

    return pl.pallas_call(
        body,
        grid=(S, NB),
        in_specs=[
            pl.BlockSpec((2, 1, NT, 16), lambda s, nn: (0, s, nn, 0)),
            pl.BlockSpec((NT, 1), lambda s, nn: (nn, 0)),
        ],
        out_specs=[pl.BlockSpec((1, NT, 16), lambda s, nn: (s, nn, 0))] * 2,
        out_shape=[jax.ShapeDtypeStruct((S, N, 16), jnp.float32)] * 2,
    )(parts, dis2)


def _combine_b(parts, tx0, tx1, dis2, W, bvec, S):
    def body(p_ref, x0_ref, x1_ref, d_ref, w_ref, b_ref, o_ref):
        a2 = p_ref[0, 0] + p_ref[1, 0]
        x0 = x0_ref[0]
        x1 = x1_ref[0]
        x2 = -2.0 * d_ref[:] * a2 - x0
        dn = (((1,), (0,)), ((), ()))
        o = (lax.dot_general(x0, w_ref[0], dn, preferred_element_type=jnp.float32)
             + lax.dot_general(x1, w_ref[1], dn, preferred_element_type=jnp.float32)
             + lax.dot_general(x2, w_ref[2], dn, preferred_element_type=jnp.float32)
             + b_ref[:])
        o_ref[0] = jnp.maximum(o, 0.0)

    return pl.pallas_call(
        body,
        grid=(S, NB),
        in_specs=[
            pl.BlockSpec((2, 1, NT, 16), lambda s, nn: (0, s, nn, 0)),
            pl.BlockSpec((1, NT, 16), lambda s, nn: (s, nn, 0)),
            pl.BlockSpec((1, NT, 16), lambda s, nn: (s, nn, 0)),
            pl.BlockSpec((NT, 1), lambda s, nn: (nn, 0)),
            pl.BlockSpec((3, 16, 16), lambda s, nn: (0, 0, 0)),
            pl.BlockSpec((1, 16), lambda s, nn: (0, 0)),
        ],
        out_specs=pl.BlockSpec((1, NT, 16), lambda s, nn: (s, nn, 0)),
        out_shape=jax.ShapeDtypeStruct((S, N, 16), jnp.float32),
    )(parts, tx0, tx1, dis2, W, bvec)


def _bn_elu(t4, g, b):
    Bb, To, _, C = t4.shape
    x3 = t4.reshape(Bb * To, N, C)
    M = Bb * To
    inv_cnt = 1.0 / (M * C)

    def body(x_ref, g_ref, b_ref, o_ref):
        x = x_ref[...]
        s = jnp.sum(jnp.sum(x, axis=2, keepdims=True), axis=0, keepdims=True)
        s2 = jnp.sum(jnp.sum(x * x, axis=2, keepdims=True), axis=0, keepdims=True)
        m = s * inv_cnt
        v = s2 * inv_cnt - m * m
        xn = (x - m) * lax.rsqrt(v + 1e-5)
        xn = xn * g_ref[:][None] + b_ref[:][None]
        o_ref[...] = jnp.where(xn > 0.0, xn, jnp.exp(xn) - 1.0)

    out = pl.pallas_call(
        body,
        grid=(NB,),
        in_specs=[
            pl.BlockSpec((M, NT, C), lambda nn: (0, nn, 0)),
            pl.BlockSpec((NT, 1), lambda nn: (nn, 0)),
            pl.BlockSpec((NT, 1), lambda nn: (nn, 0)),
        ],
        out_specs=pl.BlockSpec((M, NT, C), lambda nn: (0, nn, 0)),
        out_shape=jax.ShapeDtypeStruct((M, N, C), jnp.float32),
    )(x3, g.reshape(N, 1), b.reshape(N, 1))
    return out.reshape(Bb, To, N, C)


def _fc(h2, wpad):
    K = h2.shape[1]
    KC = 128000
    G = K // KC

    def body(x_ref, w_ref, o_ref):
        @pl.when(pl.program_id(0) == 0)
        def _():
            o_ref[...] = jnp.zeros((8, 16), jnp.float32)

        y = lax.dot_general(x_ref[...], w_ref[...], (((1,), (1,)), ((), ())),
                            preferred_element_type=jnp.float32)
        o_ref[...] = o_ref[...] + jnp.concatenate(
            [y, jnp.zeros((6, 16), jnp.float32)], axis=0)

    return pl.pallas_call(
        body,
        grid=(G,),
        in_specs=[
            pl.BlockSpec((B, KC), lambda k: (0, k)),
            pl.BlockSpec((16, KC), lambda k: (0, k)),
        ],
        out_specs=pl.BlockSpec((8, 16), lambda k: (0, 0)),
        out_shape=jax.ShapeDtypeStruct((8, 16), jnp.float32),
    )(h2, wpad)


def _cheb_block(x4, W, bvec, dis2, y4, srcf, dstp, zeros):
    Bb, Tc, _, F = x4.shape
    S = Bb * Tc
    tx0 = x4.reshape(S, N, F)
    y0 = y4.reshape(S, N, F)
    parts1 = _scatter(y0, srcf[:S], dstp, zeros, S)
    tx1, y1 = _combine_a(parts1, dis2, S)
    parts2 = _scatter(y1, srcf[:S], dstp, zeros, S)
    out = _combine_b(parts2, tx0, tx1, dis2, W, bvec, S)
    return out.reshape(Bb, Tc, N, F)


def kernel(x, edge_index, st1_tc1_w, st1_tc1_b, st1_cheb_w, st1_cheb_b,
           st1_tc2_w, st1_tc2_b, st1_bn_g, st1_bn_b, st2_tc1_w, st2_tc1_b,
           st2_cheb_w, st2_cheb_b, st2_tc2_w, st2_tc2_b, st2_bn_g, st2_bn_b,
           fc_w, fc_b):
    ei = edge_index.astype(jnp.int32)
    src, dst = ei[0], ei[1]
    ew = src != dst
    deg = jnp.zeros((N,), jnp.float32).at[src].add(ew.astype(jnp.float32))
    dis = jnp.where(deg > 0, lax.rsqrt(jnp.where(deg > 0, deg, 1.0)), 0.0)
    dis2 = dis.reshape(N, 1)

    pad = EP - E
    src_p = jnp.concatenate([src, jnp.zeros((pad,), jnp.int32)])
    dst_p = jnp.concatenate([jnp.where(ew, dst, N), jnp.full((pad,), N, jnp.int32)])
    dstp = dst_p.reshape(NW, NCH, CW)
    S1 = B * (T - 2)
    srcf = (src_p[None, :] + (jnp.arange(S1, dtype=jnp.int32) * N)[:, None])
    srcf = srcf.reshape(S1, NW, NCH, CW)
    zeros = jnp.zeros((STRIPE, 16), jnp.float32)

    t1, y1 = _tconv(x, st1_tc1_w, st1_tc1_b, dis2)
    c1 = _cheb_block(t1, st1_cheb_w, st1_cheb_b.reshape(1, HID), dis2, y1,
                     srcf, dstp, zeros)
    t2, _ = _tconv(c1, st1_tc2_w, st1_tc2_b, None)
    h = _bn_elu(t2, st1_bn_g, st1_bn_b)

    t3, y3 = _tconv(h, st2_tc1_w, st2_tc1_b, dis2)
    c2 = _cheb_block(t3, st2_cheb_w, st2_cheb_b.reshape(1, HID), dis2, y3,
                     srcf, dstp, zeros)
    t4, _ = _tconv(c2, st2_tc2_w, st2_tc2_b, None)
    h = _bn_elu(t4, st2_bn_g, st2_bn_b)

    h2 = h.reshape(B, -1)
    wpad = jnp.concatenate(
        [fc_w, jnp.zeros((16 - NCLS, fc_w.shape[1]), jnp.float32)], axis=0)
    logits = _fc(h2, wpad)
    return logits[:B, :NCLS] + fc_b[None, :]


# speedup vs baseline: 11.8065x; 1.0827x over previous
import functools

import jax
import jax.numpy as jnp
from jax import lax
from jax.experimental import pallas as pl
from jax.experimental.pallas import tpu as pltpu
from jax.experimental.pallas import tpu_sc as plsc

N = 10000
B = 2
T = 12
E = 160000
HID = 16
OUT = 64
NCLS = 10

NT = 1000
NB = N // NT
NP = 10112
NW = 32
STRIPE = NP // 16
EP = 163840
EPW = EP // NW
CW = 512
NCH = EPW // CW


def _make_scatter(S):
    mesh = plsc.VectorSubcoreMesh(core_axis_name="c", subcore_axis_name="s")

    @functools.partial(
        pl.kernel,
        mesh=mesh,
        compiler_params=pltpu.CompilerParams(use_tc_tiling_on_sc=False),
        out_type=jax.ShapeDtypeStruct((2, S, NP, 16), jnp.float32),
        scratch_types=[
            pltpu.VMEM((NCH, CW), jnp.int32),
            pltpu.VMEM((NCH, CW), jnp.int32),
            pltpu.VMEM((NCH, CW, 16), jnp.float32),
            pltpu.VMEM((STRIPE, 16), jnp.float32),
            pltpu.VMEM_SHARED((NP, 16), jnp.float32),
            pltpu.SemaphoreType.DMA,
            pltpu.SemaphoreType.DMA,
        ],
    )
    def scat(yf, srcf, dstp, zeros_hbm, parts, src_v, dst_v, rows_v, zeros_v,
             acc, sem0, sem1):
        sems = (sem0, sem1)
        cid = lax.axis_index("c")
        sid = lax.axis_index("s")
        wid = cid * 16 + sid
        pltpu.sync_copy(dstp.at[wid], dst_v)
        pltpu.sync_copy(zeros_hbm, zeros_v)
        for s in range(S):
            pltpu.sync_copy(zeros_v, acc.at[pl.ds(sid * STRIPE, STRIPE)])
            pltpu.sync_copy(srcf.at[s, wid], src_v)
            plsc.subcore_barrier()

            pending = pltpu.async_copy(yf.at[src_v.at[0]], rows_v.at[0], sems[0])
            for j in range(NCH):
                nxt = None
                if j + 1 < NCH:
                    nxt = pltpu.async_copy(yf.at[src_v.at[j + 1]],
                                           rows_v.at[j + 1], sems[(j + 1) % 2])
                pending.wait()
                pltpu.sync_copy(rows_v.at[j], acc.at[dst_v.at[j]], add=True)
                pending = nxt
            plsc.subcore_barrier()
            pltpu.sync_copy(acc.at[pl.ds(sid * STRIPE, STRIPE)],
                            parts.at[cid, s, pl.ds(sid * STRIPE, STRIPE)])
            plsc.subcore_barrier()

    return scat


def _scatter(y3, srcf, dstp, zeros, S):
    yf = y3.reshape(S * N, 16)
    return _make_scatter(S)(yf, srcf, dstp, zeros)


def _tconv(X, w, b, dis2):
    Bb, Tc, _, C = X.shape
    Cout = w.shape[1]
    To = Tc - 2
    wk = jnp.transpose(w[:, :, :, 0, :], (0, 3, 2, 1))
    want_y = dis2 is not None

    def body(x_ref, w_ref, b_ref, d_ref, t_ref, *maybe_y):
        t = pl.program_id(2)
        xs = [jnp.reshape(x_ref[0, pl.ds(t + k, 1)], (NT, C)) for k in range(3)]

        def conv(g):
            acc = jnp.zeros((NT, Cout), jnp.float32)
            for k in range(3):
                acc = acc + lax.dot_general(
                    xs[k], w_ref[g, k], (((1,), (0,)), ((), ())),
                    preferred_element_type=jnp.float32)
            return acc + b_ref[g][None, :]

        P = conv(0)
        Q = jax.nn.sigmoid(conv(1))
        R = conv(2)
        H = jnp.maximum(P * Q + R, 0.0)
        t_ref[0, 0] = H
        if maybe_y:
            maybe_y[0][0, 0] = d_ref[:] * H

    out_shapes = [jax.ShapeDtypeStruct((Bb, To, N, Cout), jnp.float32)]
    if want_y:
        out_shapes.append(jax.ShapeDtypeStruct((Bb, To, N, Cout), jnp.float32))
    d_in = dis2 if want_y else jnp.zeros((N, 1), jnp.float32)
    res = pl.pallas_call(
        body,
        grid=(Bb, NB, To),
        in_specs=[
            pl.BlockSpec((1, Tc, NT, C), lambda bb, nn, tt: (bb, 0, nn, 0)),
            pl.BlockSpec((3, 3, C, Cout), lambda bb, nn, tt: (0, 0, 0, 0)),
            pl.BlockSpec((3, Cout), lambda bb, nn, tt: (0, 0)),
            pl.BlockSpec((NT, 1), lambda bb, nn, tt: (nn, 0)),
        ],
        out_specs=[pl.BlockSpec((1, 1, NT, Cout), lambda bb, nn, tt: (bb, tt, nn, 0))
                   for _ in out_shapes],
        out_shape=out_shapes,
    )(X, wk, b, d_in)
    return res if want_y else (res[0], None)


def _combine_a(parts, dis2, S):
    def body(p_ref, d_ref, tx1_ref, y1_ref):
        a = p_ref[0, 0] + p_ref[1, 0]
        d = d_ref[:]
        tx1 = -d * a
        tx1_ref[0] = tx1
        y1_ref[0] = d * tx1

    return pl.pallas_call(
        body,
        grid=(S, NB),
        in_specs=[
            pl.BlockSpec((2, 1, NT, 16), lambda s, nn: (0, s, nn, 0)),
            pl.BlockSpec((NT, 1), lambda s, nn: (nn, 0)),
        ],
        out_specs=[pl.BlockSpec((1, NT, 16), lambda s, nn: (s, nn, 0))] * 2,
        out_shape=[jax.ShapeDtypeStruct((S, N, 16), jnp.float32)] * 2,
    )(parts, dis2)


def _combine_b(parts, tx0, tx1, dis2, W, bvec, S):
    def body(p_ref, x0_ref, x1_ref, d_ref, w_ref, b_ref, o_ref):
        a2 = p_ref[0, 0] + p_ref[1, 0]
        x0 = x0_ref[0]
        x1 = x1_ref[0]
        x2 = -2.0 * d_ref[:] * a2 - x0
        dn = (((1,), (0,)), ((), ()))
        o = (lax.dot_general(x0, w_ref[0], dn, preferred_element_type=jnp.float32)
             + lax.dot_general(x1, w_ref[1], dn, preferred_element_type=jnp.float32)
             + lax.dot_general(x2, w_ref[2], dn, preferred_element_type=jnp.float32)
             + b_ref[:])
        o_ref[0] = jnp.maximum(o, 0.0)

    return pl.pallas_call(
        body,
        grid=(S, NB),
        in_specs=[
            pl.BlockSpec((2, 1, NT, 16), lambda s, nn: (0, s, nn, 0)),
            pl.BlockSpec((1, NT, 16), lambda s, nn: (s, nn, 0)),
            pl.BlockSpec((1, NT, 16), lambda s, nn: (s, nn, 0)),
            pl.BlockSpec((NT, 1), lambda s, nn: (nn, 0)),
            pl.BlockSpec((3, 16, 16), lambda s, nn: (0, 0, 0)),
            pl.BlockSpec((1, 16), lambda s, nn: (0, 0)),
        ],
        out_specs=pl.BlockSpec((1, NT, 16), lambda s, nn: (s, nn, 0)),
        out_shape=jax.ShapeDtypeStruct((S, N, 16), jnp.float32),
    )(parts, tx0, tx1, dis2, W, bvec)


def _bn_elu(t4, g, b):
    Bb, To, _, C = t4.shape
    x3 = t4.reshape(Bb * To, N, C)
    M = Bb * To
    inv_cnt = 1.0 / (M * C)

    def body(x_ref, g_ref, b_ref, o_ref):
        x = x_ref[...]
        s = jnp.sum(jnp.sum(x, axis=2, keepdims=True), axis=0, keepdims=True)
        s2 = jnp.sum(jnp.sum(x * x, axis=2, keepdims=True), axis=0, keepdims=True)
        m = s * inv_cnt
        v = s2 * inv_cnt - m * m
        xn = (x - m) * lax.rsqrt(v + 1e-5)
        xn = xn * g_ref[:][None] + b_ref[:][None]
        o_ref[...] = jnp.where(xn > 0.0, xn, jnp.exp(xn) - 1.0)

    out = pl.pallas_call(
        body,
        grid=(NB,),
        in_specs=[
            pl.BlockSpec((M, NT, C), lambda nn: (0, nn, 0)),
            pl.BlockSpec((NT, 1), lambda nn: (nn, 0)),
            pl.BlockSpec((NT, 1), lambda nn: (nn, 0)),
        ],
        out_specs=pl.BlockSpec((M, NT, C), lambda nn: (0, nn, 0)),
        out_shape=jax.ShapeDtypeStruct((M, N, C), jnp.float32),
    )(x3, g.reshape(N, 1), b.reshape(N, 1))
    return out.reshape(Bb, To, N, C)


def _fc(h2, wpad):
    K = h2.shape[1]
    KC = 128000
    G = K // KC

    def body(x_ref, w_ref, o_ref):
        @pl.when(pl.program_id(0) == 0)
        def _():
            o_ref[...] = jnp.zeros((8, 16), jnp.float32)

        y = lax.dot_general(x_ref[...], w_ref[...], (((1,), (1,)), ((), ())),
                            preferred_element_type=jnp.float32)
        o_ref[...] = o_ref[...] + jnp.concatenate(
            [y, jnp.zeros((6, 16), jnp.float32)], axis=0)

    return pl.pallas_call(
        body,
        grid=(G,),
        in_specs=[
            pl.BlockSpec((B, KC), lambda k: (0, k)),
            pl.BlockSpec((16, KC), lambda k: (0, k)),
        ],
        out_specs=pl.BlockSpec((8, 16), lambda k: (0, 0)),
        out_shape=jax.ShapeDtypeStruct((8, 16), jnp.float32),
    )(h2, wpad)


def _cheb_block(x4, W, bvec, dis2, y4, srcf, dstp, zeros):
    Bb, Tc, _, F = x4.shape
    S = Bb * Tc
    tx0 = x4.reshape(S, N, F)
    y0 = y4.reshape(S, N, F)
    parts1 = _scatter(y0, srcf[:S], dstp, zeros, S)
    tx1, y1 = _combine_a(parts1, dis2, S)
    parts2 = _scatter(y1, srcf[:S], dstp, zeros, S)
    out = _combine_b(parts2, tx0, tx1, dis2, W, bvec, S)
    return out.reshape(Bb, Tc, N, F)


def kernel(x, edge_index, st1_tc1_w, st1_tc1_b, st1_cheb_w, st1_cheb_b,
           st1_tc2_w, st1_tc2_b, st1_bn_g, st1_bn_b, st2_tc1_w, st2_tc1_b,
           st2_cheb_w, st2_cheb_b, st2_tc2_w, st2_tc2_b, st2_bn_g, st2_bn_b,
           fc_w, fc_b):
    ei = edge_index.astype(jnp.int32)
    src, dst = ei[0], ei[1]
    ew = src != dst
    deg = jnp.zeros((N,), jnp.float32).at[src].add(ew.astype(jnp.float32))
    dis = jnp.where(deg > 0, lax.rsqrt(jnp.where(deg > 0, deg, 1.0)), 0.0)
    dis2 = dis.reshape(N, 1)

    pad = EP - E
    src_p = jnp.concatenate([src, jnp.zeros((pad,), jnp.int32)])
    dst_p = jnp.concatenate([jnp.where(ew, dst, N), jnp.full((pad,), N, jnp.int32)])
    dstp = dst_p.reshape(NW, NCH, CW)
    S1 = B * (T - 2)
    srcf = (src_p[None, :] + (jnp.arange(S1, dtype=jnp.int32) * N)[:, None])
    srcf = srcf.reshape(S1, NW, NCH, CW)
    zeros = jnp.zeros((STRIPE, 16), jnp.float32)

    t1, y1 = _tconv(x, st1_tc1_w, st1_tc1_b, dis2)
    c1 = _cheb_block(t1, st1_cheb_w, st1_cheb_b.reshape(1, HID), dis2, y1,
                     srcf, dstp, zeros)
    t2, _ = _tconv(c1, st1_tc2_w, st1_tc2_b, None)
    h = _bn_elu(t2, st1_bn_g, st1_bn_b)

    t3, y3 = _tconv(h, st2_tc1_w, st2_tc1_b, dis2)
    c2 = _cheb_block(t3, st2_cheb_w, st2_cheb_b.reshape(1, HID), dis2, y3,
                     srcf, dstp, zeros)
    t4, _ = _tconv(c2, st2_tc2_w, st2_tc2_b, None)
    h = _bn_elu(t4, st2_bn_g, st2_bn_b)

    h2 = h.reshape(B, -1)
    wpad = jnp.concatenate(
        [fc_w, jnp.zeros((16 - NCLS, fc_w.shape[1]), jnp.float32)], axis=0)
    logits = _fc(h2, wpad)
    return logits[:B, :NCLS] + fc_b[None, :]
